# baseline (speedup 1.0000x reference)
import jax
import jax.numpy as jnp
from jax.experimental import pallas as pl
from jax.experimental.pallas import tpu as pltpu

_B = 1024
_D = 16
_A = 100000

_BB = 256
_TA = 2048
_NB = _B // _BB
_NA = (_A + _TA - 1) // _TA

_NEG = -1e30
_IMAX = 2**31 - 1


def _gfn_kernel(s_ref, w_ref, b_ref, u_ref, act_ref, lp_ref,
                mz_ref, idx_ref, ml_ref, ss_ref, bl_ref):
    a = pl.program_id(1)

    @pl.when(a == 0)
    def _init():
        mz_ref[:] = jnp.full((_BB,), _NEG, jnp.float32)
        idx_ref[:] = jnp.zeros((_BB,), jnp.int32)
        ml_ref[:] = jnp.full((_BB,), _NEG, jnp.float32)
        ss_ref[:] = jnp.zeros((_BB,), jnp.float32)
        bl_ref[:] = jnp.zeros((_BB,), jnp.float32)

    logits = jnp.dot(s_ref[:], w_ref[:], preferred_element_type=jnp.float32)
    logits = logits + b_ref[:][None, :]

    cols = a * _TA + jax.lax.broadcasted_iota(jnp.int32, (_BB, _TA), 1)
    valid = cols < _A
    lgt = jnp.where(valid, logits, _NEG)

    g = -jnp.log(-jnp.log(u_ref[:]))
    z = jnp.where(valid, logits + g, _NEG)

    zmax = jnp.max(z, axis=-1)
    lidx = jnp.min(jnp.where(z == zmax[:, None], cols, _IMAX), axis=-1)
    la = jnp.sum(jnp.where(cols == lidx[:, None], lgt, 0.0), axis=-1)

    upd = zmax > mz_ref[:]
    idx_ref[:] = jnp.where(upd, lidx, idx_ref[:])
    bl_ref[:] = jnp.where(upd, la, bl_ref[:])
    mz_ref[:] = jnp.maximum(mz_ref[:], zmax)

    tml = jnp.max(lgt, axis=-1)
    m_old = ml_ref[:]
    m_new = jnp.maximum(m_old, tml)
    p = jnp.exp(lgt - m_new[:, None])
    ss_ref[:] = ss_ref[:] * jnp.exp(m_old - m_new) + jnp.sum(p, axis=-1)
    ml_ref[:] = m_new

    @pl.when(a == _NA - 1)
    def _fin():
        act_ref[:] = idx_ref[:]
        lp_ref[:] = bl_ref[:] - ml_ref[:] - jnp.log(ss_ref[:])


def kernel(s, u, W, b):
    actions, log_prob = pl.pallas_call(
        _gfn_kernel,
        grid=(_NB, _NA),
        in_specs=[
            pl.BlockSpec((_BB, _D), lambda i, j: (i, 0)),
            pl.BlockSpec((_D, _TA), lambda i, j: (0, j)),
            pl.BlockSpec((_TA,), lambda i, j: (j,)),
            pl.BlockSpec((_BB, _TA), lambda i, j: (i, j)),
        ],
        out_specs=[
            pl.BlockSpec((_BB,), lambda i, j: (i,)),
            pl.BlockSpec((_BB,), lambda i, j: (i,)),
        ],
        out_shape=[
            jax.ShapeDtypeStruct((_B,), jnp.int32),
            jax.ShapeDtypeStruct((_B,), jnp.float32),
        ],
        scratch_shapes=[
            pltpu.VMEM((_BB,), jnp.float32),
            pltpu.VMEM((_BB,), jnp.int32),
            pltpu.VMEM((_BB,), jnp.float32),
            pltpu.VMEM((_BB,), jnp.float32),
            pltpu.VMEM((_BB,), jnp.float32),
        ],
        compiler_params=pltpu.CompilerParams(
            dimension_semantics=("parallel", "arbitrary"),
        ),
    )(s, W, b, u)
    return (actions, log_prob)
